# baseline (speedup 1.0000x reference)
import functools

import jax
import jax.numpy as jnp
from jax import lax
from jax.experimental import pallas as pl
from jax.experimental.pallas import tpu as pltpu
from jax.experimental.pallas import tpu_sc as plsc

N_NODES = 10000
N_PAD = 10240
D_FEAT = 128
N_EDGES = 320000
BLK = 128
E_PAD = 327680
NBLK = E_PAD // BLK
NCORE = 2
NSUB = 16
NB = NBLK // (NCORE * NSUB)
ZCHUNK = 128

_mesh = plsc.VectorSubcoreMesh(core_axis_name="c", subcore_axis_name="s")


@functools.partial(
    pl.kernel,
    mesh=_mesh,
    out_type=jax.ShapeDtypeStruct((NCORE, N_PAD, D_FEAT), jnp.float32),
    scratch_types=[
        pltpu.VMEM((BLK,), jnp.int32),
        pltpu.VMEM((BLK,), jnp.int32),
        pltpu.VMEM((BLK,), jnp.int32),
        pltpu.VMEM((BLK,), jnp.int32),
        pltpu.VMEM((BLK, D_FEAT), jnp.float32),
        pltpu.VMEM((BLK, D_FEAT), jnp.float32),
        pltpu.VMEM_SHARED((N_PAD, D_FEAT), jnp.float32),
        pltpu.SemaphoreType.DMA,
        pltpu.SemaphoreType.DMA,
        pltpu.SemaphoreType.DMA,
        pltpu.SemaphoreType.DMA,
    ],
)
def _sc_agg(x_hbm, src_hbm, dst_hbm, out_hbm,
            srcb0, srcb1, dstb0, dstb1, rows0, rows1, acc_sh,
            isem0, isem1, gsem0, gsem1):
    c = lax.axis_index("c")
    s = lax.axis_index("s")
    srcb = (srcb0, srcb1)
    dstb = (dstb0, dstb1)
    rows = (rows0, rows1)
    isem = (isem0, isem1)
    gsem = (gsem0, gsem1)
    rpt = N_PAD // NSUB
    start = (c * (NBLK // NCORE) + s * NB) * BLK

    def idx_start(j, b):
        pltpu.async_copy(src_hbm.at[pl.ds(start + j * BLK, BLK)], srcb[b], isem[b])
        pltpu.async_copy(dst_hbm.at[pl.ds(start + j * BLK, BLK)], dstb[b], isem[b])

    def idx_wait(j, b):
        pltpu.make_async_copy(
            src_hbm.at[pl.ds(start + j * BLK, BLK)], srcb[b], isem[b]).wait()
        pltpu.make_async_copy(
            dst_hbm.at[pl.ds(start + j * BLK, BLK)], dstb[b], isem[b]).wait()

    def zrow(i, carry):
        for j in range(D_FEAT // 16):
            rows0[i, pl.ds(j * 16, 16)] = jnp.zeros((16,), jnp.float32)
        return carry

    lax.fori_loop(0, ZCHUNK, zrow, 0)
    for k in range(rpt // ZCHUNK):
        pltpu.sync_copy(
            rows0, acc_sh.at[pl.ds(s * rpt + k * ZCHUNK, ZCHUNK)])
    plsc.subcore_barrier()

    idx_start(0, 0)
    idx_start(1, 1)
    idx_wait(0, 0)
    pltpu.async_copy(x_hbm.at[srcb0], rows0, gsem0)

    def step(j, b, prefetch):
        o = b ^ 1
        pltpu.make_async_copy(x_hbm.at[srcb[b]], rows[b], gsem[b]).wait()
        if prefetch:
            idx_wait(j + 1, o)
            pltpu.async_copy(x_hbm.at[srcb[o]], rows[o], gsem[o])
        pltpu.sync_copy(rows[b], acc_sh.at[dstb[b]], add=True)
        if prefetch:
            idx_start(j + 2, b)

    def pair(g, carry):
        for b in range(2):
            step(2 * g + b, b, True)
        return carry

    lax.fori_loop(0, NB // 2 - 1, pair, 0)
    o = NB % 2
    pltpu.make_async_copy(x_hbm.at[srcb0], rows0, gsem0).wait()
    idx_wait(NB - 1, 1)
    pltpu.async_copy(x_hbm.at[srcb1], rows1, gsem1)
    pltpu.sync_copy(rows0, acc_sh.at[dstb0], add=True)
    pltpu.make_async_copy(x_hbm.at[srcb1], rows1, gsem1).wait()
    pltpu.sync_copy(rows1, acc_sh.at[dstb1], add=True)

    plsc.subcore_barrier()
    pltpu.sync_copy(
        acc_sh.at[pl.ds(s * rpt, rpt)],
        out_hbm.at[c, pl.ds(s * rpt, rpt)])


def _add_body(a_ref, b_ref, o_ref):
    o_ref[...] = a_ref[...] + b_ref[...]


_tc_add = pl.pallas_call(
    _add_body,
    out_shape=jax.ShapeDtypeStruct((N_PAD, D_FEAT), jnp.float32),
    grid=(10,),
    in_specs=[
        pl.BlockSpec((N_PAD // 10, D_FEAT), lambda i: (i, 0)),
        pl.BlockSpec((N_PAD // 10, D_FEAT), lambda i: (i, 0)),
    ],
    out_specs=pl.BlockSpec((N_PAD // 10, D_FEAT), lambda i: (i, 0)),
)


def kernel(x, edge_index):
    src = edge_index[0].astype(jnp.int32)
    dst = edge_index[1].astype(jnp.int32)
    pad = E_PAD - N_EDGES
    src1 = jnp.pad(src, (0, pad), constant_values=N_NODES)
    dst1 = jnp.pad(dst, (0, pad), constant_values=N_NODES)
    xp = jnp.pad(x, ((0, N_PAD - N_NODES), (0, 0)))
    parts = _sc_agg(xp, src1, dst1)
    out = _tc_add(parts[0], parts[1])
    return out[:N_NODES]
